# baseline (speedup 1.0000x reference)
import functools

import jax
import jax.numpy as jnp
from jax import lax
from jax.experimental import pallas as pl
from jax.experimental.pallas import tpu as pltpu
from jax.experimental.pallas import tpu_sc as plsc

N_FIELDS = 26
VOCAB = 100000
EMB_DIM = 16
LANES = 16
NUM_CORES = 2
NUM_SUBCORES = 16
NW = NUM_CORES * NUM_SUBCORES

IDX_BLK = 128
SUPER = 13


def _sc_gather(tbl_flat, idx2d):
    R = idx2d.shape[0]
    rows_per_w = R // NW
    n_super = rows_per_w // SUPER

    mesh = plsc.VectorSubcoreMesh(core_axis_name="c", subcore_axis_name="s")

    @functools.partial(
        pl.kernel,
        mesh=mesh,
        out_type=jax.ShapeDtypeStruct((R * IDX_BLK, EMB_DIM), jnp.float32),
        scratch_types=[
            pltpu.VMEM((rows_per_w, IDX_BLK), jnp.int32),
            pltpu.VMEM((2, SUPER * IDX_BLK, EMB_DIM), jnp.float32),
            pltpu.SemaphoreType.DMA,
            pltpu.SemaphoreType.DMA,
        ],
        compiler_params=pltpu.CompilerParams(use_tc_tiling_on_sc=False),
    )
    def k(idx_hbm, tbl_hbm, out_hbm, idx_v, rows_v, sem0, sem1):
        wid = lax.axis_index("s") * NUM_CORES + lax.axis_index("c")
        rbase = wid * rows_per_w
        pltpu.sync_copy(idx_hbm.at[pl.ds(rbase, rows_per_w)], idx_v)

        lane = lax.iota(jnp.int32, LANES)
        sems = (sem0, sem1)

        def fixup(c):
            def fix_row(r0, _):
                r = c * SUPER + r0
                def fix_vec(i, _):
                    pos = (rbase + r) * IDX_BLK + i * LANES
                    f = lax.rem(pos + lane, N_FIELDS)
                    sl = pl.ds(i * LANES, LANES)
                    idx_v[r, sl] = idx_v[r, sl] + f * VOCAB
                    return 0
                return lax.fori_loop(0, IDX_BLK // LANES, fix_vec, 0)
            lax.fori_loop(0, SUPER, fix_row, 0)

        def fire(c, buf):
            def f(j, _):
                pltpu.async_copy(
                    tbl_hbm.at[idx_v.at[c * SUPER + j]],
                    rows_v.at[buf, pl.ds(j * IDX_BLK, IDX_BLK)],
                    sems[buf],
                )
                return 0
            lax.fori_loop(0, SUPER, f, 0)

        def drain_store(c, buf):
            def d(j, _):
                pltpu.make_async_copy(
                    tbl_hbm.at[idx_v.at[c * SUPER + j]],
                    rows_v.at[buf, pl.ds(j * IDX_BLK, IDX_BLK)],
                    sems[buf],
                ).wait()
                return 0
            lax.fori_loop(0, SUPER, d, 0)
            pltpu.sync_copy(
                rows_v.at[buf],
                out_hbm.at[pl.ds((rbase + c * SUPER) * IDX_BLK,
                                 SUPER * IDX_BLK)],
            )

        fixup(0)
        fire(0, 0)
        for c in range(1, n_super):
            fixup(c)
            fire(c, c % 2)
            drain_store(c - 1, (c - 1) % 2)
        drain_store(n_super - 1, (n_super - 1) % 2)

    return k(idx2d, tbl_flat)


def _tc_mlp(numerical, emb, W1n, W1e, b1, W2, b2, W3, b3, bm=4096):
    B = numerical.shape[0]
    nd = numerical.shape[1]
    ed = emb.shape[1]
    h1 = W1n.shape[1]
    h2 = W2.shape[1]

    def body(num_ref, emb_ref, w1n_ref, w1e_ref, b1_ref, w2_ref, b2_ref,
             w3_ref, b3_ref, out_ref):
        h = jnp.dot(num_ref[...], w1n_ref[...],
                    preferred_element_type=jnp.float32)
        h = h + jnp.dot(emb_ref[...], w1e_ref[...],
                        preferred_element_type=jnp.float32)
        h = jnp.maximum(h + b1_ref[...], 0.0)
        h = jnp.maximum(jnp.dot(h, w2_ref[...],
                                preferred_element_type=jnp.float32)
                        + b2_ref[...], 0.0)
        out_ref[...] = jnp.dot(h, w3_ref[...],
                               preferred_element_type=jnp.float32) + b3_ref[...]

    return pl.pallas_call(
        body,
        grid=(B // bm,),
        in_specs=[
            pl.BlockSpec((bm, nd), lambda i: (i, 0)),
            pl.BlockSpec((bm, ed), lambda i: (i, 0)),
            pl.BlockSpec((nd, h1), lambda i: (0, 0)),
            pl.BlockSpec((ed, h1), lambda i: (0, 0)),
            pl.BlockSpec((1, h1), lambda i: (0, 0)),
            pl.BlockSpec((h1, h2), lambda i: (0, 0)),
            pl.BlockSpec((1, h2), lambda i: (0, 0)),
            pl.BlockSpec((h2, 1), lambda i: (0, 0)),
            pl.BlockSpec((1, 1), lambda i: (0, 0)),
        ],
        out_specs=pl.BlockSpec((bm, 1), lambda i: (i, 0)),
        out_shape=jax.ShapeDtypeStruct((B, 1), jnp.float32),
        compiler_params=pltpu.CompilerParams(
            dimension_semantics=("arbitrary",)),
    )(numerical, emb, W1n, W1e, b1, W2, b2, W3, b3)


def kernel(numerical, categorical, tables, W1, b1, W2, b2, W3, b3):
    B, nf = categorical.shape
    nd = numerical.shape[1]
    tbl_flat = tables.reshape(nf * VOCAB, EMB_DIM)
    idx2d = categorical.astype(jnp.int32).reshape(-1, IDX_BLK)
    emb = _sc_gather(tbl_flat, idx2d).reshape(B, nf * EMB_DIM)
    out = _tc_mlp(numerical, emb,
                  W1[:nd], W1[nd:], b1.reshape(1, -1),
                  W2, b2.reshape(1, -1), W3, b3.reshape(1, 1))
    return out


# speedup vs baseline: 1.0907x; 1.0907x over previous
import functools

import jax
import jax.numpy as jnp
from jax import lax
from jax.experimental import pallas as pl
from jax.experimental.pallas import tpu as pltpu
from jax.experimental.pallas import tpu_sc as plsc

N_FIELDS = 26
VOCAB = 100000
EMB_DIM = 16
LANES = 16
NUM_CORES = 2
NUM_SUBCORES = 16
NW = NUM_CORES * NUM_SUBCORES

IDX_BLK = 128
SUPER = 13


def _sc_gather(tbl_flat, idx2d):
    R = idx2d.shape[0]
    rows_per_w = R // NW
    n_super = rows_per_w // SUPER

    mesh = plsc.VectorSubcoreMesh(core_axis_name="c", subcore_axis_name="s")

    @functools.partial(
        pl.kernel,
        mesh=mesh,
        out_type=jax.ShapeDtypeStruct((R * IDX_BLK, EMB_DIM), jnp.float32),
        scratch_types=[
            pltpu.VMEM((rows_per_w, IDX_BLK), jnp.int32),
            pltpu.VMEM((2, SUPER * IDX_BLK, EMB_DIM), jnp.float32),
            pltpu.SemaphoreType.DMA,
            pltpu.SemaphoreType.DMA,
        ],
        compiler_params=pltpu.CompilerParams(use_tc_tiling_on_sc=False),
    )
    def k(idx_hbm, tbl_hbm, out_hbm, idx_v, rows_v, sem0, sem1):
        wid = lax.axis_index("s") * NUM_CORES + lax.axis_index("c")
        rbase = wid * rows_per_w
        pltpu.sync_copy(idx_hbm.at[pl.ds(rbase, rows_per_w)], idx_v)

        lane = lax.iota(jnp.int32, LANES)
        sems = (sem0, sem1)

        def fixup(c):
            def fix_row(r0, _):
                r = c * SUPER + r0
                def fix_vec(i, _):
                    pos = (rbase + r) * IDX_BLK + i * LANES
                    f = lax.rem(pos + lane, N_FIELDS)
                    sl = pl.ds(i * LANES, LANES)
                    idx_v[r, sl] = idx_v[r, sl] + f * VOCAB
                    return 0
                return lax.fori_loop(0, IDX_BLK // LANES, fix_vec, 0)
            lax.fori_loop(0, SUPER, fix_row, 0)

        def fire(c, buf):
            def f(j, _):
                pltpu.async_copy(
                    tbl_hbm.at[idx_v.at[c * SUPER + j]],
                    rows_v.at[buf, pl.ds(j * IDX_BLK, IDX_BLK)],
                    sems[buf],
                )
                return 0
            lax.fori_loop(0, SUPER, f, 0)

        def drain_store(c, buf):
            def d(j, _):
                pltpu.make_async_copy(
                    tbl_hbm.at[idx_v.at[c * SUPER + j]],
                    rows_v.at[buf, pl.ds(j * IDX_BLK, IDX_BLK)],
                    sems[buf],
                ).wait()
                return 0
            lax.fori_loop(0, SUPER, d, 0)
            pltpu.sync_copy(
                rows_v.at[buf],
                out_hbm.at[pl.ds((rbase + c * SUPER) * IDX_BLK,
                                 SUPER * IDX_BLK)],
            )

        fixup(0)
        fire(0, 0)
        for c in range(1, n_super):
            fixup(c)
            fire(c, c % 2)
            drain_store(c - 1, (c - 1) % 2)
        drain_store(n_super - 1, (n_super - 1) % 2)

    return k(idx2d, tbl_flat)


def _tc_depad(tbl3d, sk=2600):
    S = tbl3d.shape[0]
    G = S // sk

    def body(in_ref, out_ref):
        for k in range(8):
            out_ref[:, k * EMB_DIM:(k + 1) * EMB_DIM] = in_ref[:, k, :]

    return pl.pallas_call(
        body,
        grid=(G,),
        in_specs=[pl.BlockSpec((sk, 8, EMB_DIM), lambda i: (i, 0, 0))],
        out_specs=pl.BlockSpec((sk, 128), lambda i: (i, 0)),
        out_shape=jax.ShapeDtypeStruct((S, 128), jnp.float32),
        compiler_params=pltpu.CompilerParams(
            dimension_semantics=("arbitrary",)),
    )(tbl3d)


def _tc_mlp(numerical, emb, W1n, W1e, b1, W2, b2, W3, b3, bm=4096):
    B = numerical.shape[0]
    nd = numerical.shape[1]
    ed = emb.shape[1]
    h1 = W1n.shape[1]
    h2 = W2.shape[1]

    def body(num_ref, emb_ref, w1n_ref, w1e_ref, b1_ref, w2_ref, b2_ref,
             w3_ref, b3_ref, out_ref):
        h = jnp.dot(num_ref[...], w1n_ref[...],
                    preferred_element_type=jnp.float32)
        h = h + jnp.dot(emb_ref[...], w1e_ref[...],
                        preferred_element_type=jnp.float32)
        h = jnp.maximum(h + b1_ref[...], 0.0)
        h = jnp.maximum(jnp.dot(h, w2_ref[...],
                                preferred_element_type=jnp.float32)
                        + b2_ref[...], 0.0)
        out_ref[...] = jnp.dot(h, w3_ref[...],
                               preferred_element_type=jnp.float32) + b3_ref[...]

    return pl.pallas_call(
        body,
        grid=(B // bm,),
        in_specs=[
            pl.BlockSpec((bm, nd), lambda i: (i, 0)),
            pl.BlockSpec((bm, ed), lambda i: (i, 0)),
            pl.BlockSpec((nd, h1), lambda i: (0, 0)),
            pl.BlockSpec((ed, h1), lambda i: (0, 0)),
            pl.BlockSpec((1, h1), lambda i: (0, 0)),
            pl.BlockSpec((h1, h2), lambda i: (0, 0)),
            pl.BlockSpec((1, h2), lambda i: (0, 0)),
            pl.BlockSpec((h2, 1), lambda i: (0, 0)),
            pl.BlockSpec((1, 1), lambda i: (0, 0)),
        ],
        out_specs=pl.BlockSpec((bm, 1), lambda i: (i, 0)),
        out_shape=jax.ShapeDtypeStruct((B, 1), jnp.float32),
        compiler_params=pltpu.CompilerParams(
            dimension_semantics=("arbitrary",)),
    )(numerical, emb, W1n, W1e, b1, W2, b2, W3, b3)


def kernel(numerical, categorical, tables, W1, b1, W2, b2, W3, b3):
    B, nf = categorical.shape
    nd = numerical.shape[1]
    tbl_flat = _tc_depad(tables.reshape(nf * VOCAB // 8, 8, EMB_DIM)).reshape(
        nf * VOCAB, EMB_DIM)
    idx2d = categorical.astype(jnp.int32).reshape(-1, IDX_BLK)
    emb = _sc_gather(tbl_flat, idx2d).reshape(B, nf * EMB_DIM)
    out = _tc_mlp(numerical, emb,
                  W1[:nd], W1[nd:], b1.reshape(1, -1),
                  W2, b2.reshape(1, -1), W3, b3.reshape(1, 1))
    return out
